# baseline (speedup 1.0000x reference)
import functools

import jax
import jax.numpy as jnp
from jax import lax
from jax.experimental import pallas as pl
from jax.experimental.pallas import tpu as pltpu
from jax.experimental.pallas import tpu_sc as plsc

_NC = 2
_NS = 16
_NW = _NC * _NS
_L = 16


def _pe_add_tc_kernel(x_ref, e_ref, o_ref):
    o_ref[...] = x_ref[...] + e_ref[...][None, :, :]


def _tc_kernel(x, embeds):
    B, T, D = x.shape
    bt = 2048
    grid = (T // bt, B)
    return pl.pallas_call(
        _pe_add_tc_kernel,
        grid=grid,
        in_specs=[
            pl.BlockSpec((1, bt, D), lambda t, b: (b, t, 0)),
            pl.BlockSpec((bt, D), lambda t, b: (t, 0)),
        ],
        out_specs=pl.BlockSpec((1, bt, D), lambda t, b: (b, t, 0)),
        out_shape=jax.ShapeDtypeStruct((B, T, D), x.dtype),
    )(x, embeds)


def _sc_kernel(x, embeds):
    B, T, D = x.shape
    span = T // _NW
    R = 16
    n_chunks = span // R
    nvec = R * D // _L
    mesh = plsc.VectorSubcoreMesh(core_axis_name="c", subcore_axis_name="s")

    @functools.partial(
        pl.kernel,
        out_type=jax.ShapeDtypeStruct((B, T * D), x.dtype),
        mesh=mesh,
        scratch_types=[
            pltpu.VMEM((R * D,), jnp.float32),
            pltpu.VMEM((R * D,), jnp.float32),
        ],
    )
    def sc_k(x_hbm, e_hbm, out_hbm, xbuf, ebuf):
        wid = lax.axis_index("s") * _NC + lax.axis_index("c")
        t0 = wid * span

        def chunk_body(ci, _):
            o0 = (t0 + ci * R) * D
            pltpu.sync_copy(e_hbm.at[pl.ds(o0, R * D)], ebuf)

            def b_body(b, _):
                pltpu.sync_copy(x_hbm.at[b, pl.ds(o0, R * D)], xbuf)

                @plsc.parallel_loop(0, nvec, 1, unroll=8)
                def add_body(j):
                    sl = pl.ds(j * _L, _L)
                    xbuf[sl] = xbuf[sl] + ebuf[sl]

                pltpu.sync_copy(xbuf, out_hbm.at[b, pl.ds(o0, R * D)])
                return 0

            lax.fori_loop(0, B, b_body, 0)
            return 0

        lax.fori_loop(0, n_chunks, chunk_body, 0)

    out = sc_k(x.reshape(B, T * D), embeds.reshape(T * D))
    return out.reshape(B, T, D)


def kernel(x, embeds):
    return _sc_kernel(x, embeds)


# speedup vs baseline: 1.0021x; 1.0021x over previous
import functools

import jax
import jax.numpy as jnp
from jax import lax
from jax.experimental import pallas as pl
from jax.experimental.pallas import tpu as pltpu
from jax.experimental.pallas import tpu_sc as plsc

_NC = 2
_NS = 16
_NW = _NC * _NS
_L = 16


def _pe_add_tc_kernel(x_ref, e_ref, o_ref):
    o_ref[...] = x_ref[...] + e_ref[...][None, :, :]


def _tc_kernel(x, embeds):
    B, T, D = x.shape
    bt = 2048
    grid = (T // bt, B)
    return pl.pallas_call(
        _pe_add_tc_kernel,
        grid=grid,
        in_specs=[
            pl.BlockSpec((1, bt, D), lambda t, b: (b, t, 0)),
            pl.BlockSpec((bt, D), lambda t, b: (t, 0)),
        ],
        out_specs=pl.BlockSpec((1, bt, D), lambda t, b: (b, t, 0)),
        out_shape=jax.ShapeDtypeStruct((B, T, D), x.dtype),
    )(x, embeds)


def _sc_kernel(x, embeds):
    B, T, D = x.shape
    span = T // _NW
    R = 16
    n_chunks = span // R
    nvec = R * D // _L
    mesh = plsc.VectorSubcoreMesh(core_axis_name="c", subcore_axis_name="s")

    @functools.partial(
        pl.kernel,
        out_type=jax.ShapeDtypeStruct((B, T * D), x.dtype),
        mesh=mesh,
        scratch_types=[
            pltpu.VMEM((R * D,), jnp.float32),
            pltpu.VMEM((R * D,), jnp.float32),
        ],
    )
    def sc_k(x_hbm, e_hbm, out_hbm, xbuf, ebuf):
        wid = lax.axis_index("s") * _NC + lax.axis_index("c")
        t0 = wid * span

        def chunk_body(ci, _):
            o0 = (t0 + ci * R) * D
            pltpu.sync_copy(e_hbm.at[pl.ds(o0, R * D)], ebuf)

            def b_body(b, _):
                pltpu.sync_copy(x_hbm.at[b, pl.ds(o0, R * D)], xbuf)

                @plsc.parallel_loop(0, R, 1)
                def row_body(r):
                    base = r * D
                    for c in range(D // _L):
                        sl = pl.ds(base + c * _L, _L)
                        xbuf[sl] = xbuf[sl] + ebuf[sl]

                pltpu.sync_copy(xbuf, out_hbm.at[b, pl.ds(o0, R * D)])
                return 0

            lax.fori_loop(0, B, b_body, 0)
            return 0

        lax.fori_loop(0, n_chunks, chunk_body, 0)

    out = sc_k(x.reshape(B, T * D), embeds.reshape(T * D))
    return out.reshape(B, T, D)


def kernel(x, embeds):
    return _sc_kernel(x, embeds)


# speedup vs baseline: 5.0826x; 5.0717x over previous
import functools

import jax
import jax.numpy as jnp
from jax import lax
from jax.experimental import pallas as pl
from jax.experimental.pallas import tpu as pltpu
from jax.experimental.pallas import tpu_sc as plsc

_NC = 2
_NS = 16
_NW = _NC * _NS
_L = 16


def _pe_add_tc_kernel(x_ref, e_ref, o_ref):
    o_ref[...] = x_ref[...] + e_ref[...][None, :, :]


def _pe_add_tc2_kernel(x1_ref, x2_ref, e1_ref, e2_ref, o_ref):
    bt = x1_ref.shape[1]
    o_ref[:, :bt, :] = x1_ref[...] + e1_ref[...][None, :, :]
    o_ref[:, bt:, :] = x2_ref[...] + e2_ref[...][None, :, :]


def _tc2_kernel(x, embeds):
    B, T, D = x.shape
    bt = 1024
    grid = (T // (2 * bt), B)
    return pl.pallas_call(
        _pe_add_tc2_kernel,
        grid=grid,
        in_specs=[
            pl.BlockSpec((1, bt, D), lambda t, b: (b, 2 * t, 0)),
            pl.BlockSpec((1, bt, D), lambda t, b: (b, 2 * t + 1, 0)),
            pl.BlockSpec((bt, D), lambda t, b: (2 * t, 0)),
            pl.BlockSpec((bt, D), lambda t, b: (2 * t + 1, 0)),
        ],
        out_specs=pl.BlockSpec((1, 2 * bt, D), lambda t, b: (b, t, 0)),
        out_shape=jax.ShapeDtypeStruct((B, T, D), x.dtype),
    )(x, x, embeds, embeds)


def _tc_kernel(x, embeds):
    B, T, D = x.shape
    bt = 2048
    grid = (T // bt, B)
    return pl.pallas_call(
        _pe_add_tc_kernel,
        grid=grid,
        in_specs=[
            pl.BlockSpec((1, bt, D), lambda t, b: (b, t, 0)),
            pl.BlockSpec((bt, D), lambda t, b: (t, 0)),
        ],
        out_specs=pl.BlockSpec((1, bt, D), lambda t, b: (b, t, 0)),
        out_shape=jax.ShapeDtypeStruct((B, T, D), x.dtype),
    )(x, embeds)


def _sc_kernel(x, embeds):
    B, T, D = x.shape
    span = T // _NW
    R = 16
    n_chunks = span // R
    nvec = R * D // _L
    mesh = plsc.VectorSubcoreMesh(core_axis_name="c", subcore_axis_name="s")

    @functools.partial(
        pl.kernel,
        out_type=jax.ShapeDtypeStruct((B, T, D), x.dtype),
        mesh=mesh,
        scratch_types=[
            pltpu.VMEM((R, D), jnp.float32),
            pltpu.VMEM((R, D), jnp.float32),
        ],
    )
    def sc_k(x_hbm, e_hbm, out_hbm, xbuf, ebuf):
        wid = lax.axis_index("s") * _NC + lax.axis_index("c")
        t0 = wid * span

        def chunk_body(ci, _):
            tc0 = t0 + ci * R
            pltpu.sync_copy(e_hbm.at[pl.ds(tc0, R)], ebuf)

            def b_body(b, _):
                pltpu.sync_copy(x_hbm.at[b, pl.ds(tc0, R)], xbuf)

                @plsc.parallel_loop(0, R, 1)
                def row_body(r):
                    for c in range(D // _L):
                        sl = pl.ds(c * _L, _L)
                        xbuf[r, sl] = xbuf[r, sl] + ebuf[r, sl]

                pltpu.sync_copy(xbuf, out_hbm.at[b, pl.ds(tc0, R)])
                return 0

            lax.fori_loop(0, B, b_body, 0)
            return 0

        lax.fori_loop(0, n_chunks, chunk_body, 0)

    return sc_k(x, embeds)


def _sc_tail_kernel(x, embeds, t_base, k_sc):
    B, T, D = x.shape
    span = k_sc // _NW
    R = min(16, span)
    n_chunks = span // R
    mesh = plsc.VectorSubcoreMesh(core_axis_name="c", subcore_axis_name="s")

    @functools.partial(
        pl.kernel,
        out_type=jax.ShapeDtypeStruct((B, k_sc, D), x.dtype),
        mesh=mesh,
        scratch_types=[
            pltpu.VMEM((R, D), jnp.float32),
            pltpu.VMEM((R, D), jnp.float32),
        ],
    )
    def sc_k(x_hbm, e_hbm, out_hbm, xbuf, ebuf):
        wid = lax.axis_index("s") * _NC + lax.axis_index("c")
        t0 = wid * span

        def chunk_body(ci, _):
            tc0 = t0 + ci * R
            pltpu.sync_copy(e_hbm.at[pl.ds(t_base + tc0, R)], ebuf)

            def b_body(b, _):
                pltpu.sync_copy(x_hbm.at[b, pl.ds(t_base + tc0, R)], xbuf)

                def row_body(r, _):
                    for c in range(D // _L):
                        sl = pl.ds(c * _L, _L)
                        xbuf[r, sl] = xbuf[r, sl] + ebuf[r, sl]
                    return 0

                lax.fori_loop(0, R, row_body, 0)
                pltpu.sync_copy(xbuf, out_hbm.at[b, pl.ds(tc0, R)])
                return 0

            lax.fori_loop(0, B, b_body, 0)
            return 0

        lax.fori_loop(0, n_chunks, chunk_body, 0)

    return sc_k(x, embeds)


def _hybrid_kernel(x, embeds, k_sc=1024):
    B, T, D = x.shape
    T_tc = T - k_sc
    bt = 512
    tc_out = pl.pallas_call(
        _pe_add_tc_kernel,
        grid=(T_tc // bt,),
        in_specs=[
            pl.BlockSpec((B, bt, D), lambda t: (0, t, 0)),
            pl.BlockSpec((bt, D), lambda t: (t, 0)),
        ],
        out_specs=pl.BlockSpec((B, bt, D), lambda t: (0, t, 0)),
        out_shape=jax.ShapeDtypeStruct((B, T, D), x.dtype),
        name="tc_body",
    )(x, embeds)
    sc_out = _sc_tail_kernel(x, embeds, T_tc, k_sc)
    return lax.dynamic_update_slice(tc_out, sc_out, (0, T_tc, 0))


def kernel(x, embeds):
    return _tc2_kernel(x, embeds)
